# Initial kernel scaffold; baseline (speedup 1.0000x reference)
import jax
import jax.numpy as jnp
from jax.experimental import pallas as pl


def kernel(pred, target):
    raise NotImplementedError("write your pallas kernel here")


# speedup vs baseline: 11.1714x; 11.1714x over previous
import math

import jax
import jax.numpy as jnp
from jax.experimental import pallas as pl

NCLS = 722
_DECAY1 = math.exp(-0.5)
_DECAY2 = math.exp(-1.0)
_DECAY3 = math.exp(-2.0)

TOK_BLOCK = 1024


def _ce_body(pred_ref, tgt_ref, out_ref):
    i = pl.program_id(0)
    x = pred_ref[...]
    t = tgt_ref[...]
    rowmax = jnp.max(x, axis=-1, keepdims=True)
    sumexp = jnp.sum(jnp.exp(x - rowmax), axis=-1, keepdims=True)
    lse = jnp.log(sumexp) + rowmax

    col = jax.lax.broadcasted_iota(jnp.int32, x.shape, 1)
    dist = jnp.abs(col - t)
    w = jnp.where(dist == 0, 1.0,
        jnp.where(dist == 1, _DECAY1,
        jnp.where(dist == 2, _DECAY2,
        jnp.where(dist == 3, _DECAY3, 0.0))))
    wdot = jnp.sum(w * x, axis=-1, keepdims=True)
    sw = jnp.sum(w, axis=-1, keepdims=True)

    part = jnp.sum(sw * lse - wdot, axis=0, keepdims=True)

    @pl.when(i == 0)
    def _init():
        out_ref[...] = part

    @pl.when(i > 0)
    def _acc():
        out_ref[...] += part


def kernel(pred, target):
    B, T, C = pred.shape
    n_tok = B * T
    pred2 = pred.reshape(n_tok, C)
    tgt2 = target.astype(jnp.int32).reshape(n_tok, 1)
    grid = n_tok // TOK_BLOCK

    total = pl.pallas_call(
        _ce_body,
        grid=(grid,),
        in_specs=[
            pl.BlockSpec((TOK_BLOCK, C), lambda i: (i, 0)),
            pl.BlockSpec((TOK_BLOCK, 1), lambda i: (i, 0)),
        ],
        out_specs=pl.BlockSpec((1, 1), lambda i: (0, 0)),
        out_shape=jax.ShapeDtypeStruct((1, 1), jnp.float32),
    )(pred2, tgt2)
    return total[0, 0] / n_tok
